# speedup vs baseline: 1.2617x; 1.2617x over previous
import functools

import jax
import jax.numpy as jnp
from jax import lax
from jax.experimental import pallas as pl
from jax.experimental.pallas import tpu as pltpu
from jax.experimental.pallas import tpu_sc as plsc

NONEDGE_ID = 0
PADEDGE_ID = 1

ROWS, COLS = 4096, 200
N = ROWS * COLS
NC, NS, L = 2, 16, 16
NW = NC * NS
PER_W = N // NW
CHUNK = 128
CPG = 10
GROUP = CHUNK * CPG
NG = PER_W // GROUP

assert PER_W % GROUP == 0 and NG % 2 == 0

_mesh = plsc.VectorSubcoreMesh(core_axis_name="c", subcore_axis_name="s")


@functools.partial(
    pl.kernel,
    out_type=jax.ShapeDtypeStruct((N,), jnp.float32),
    mesh=_mesh,
    scratch_types=[
        pltpu.VMEM((PER_W,), jnp.int32),
        pltpu.VMEM((2, GROUP), jnp.float32),
        pltpu.VMEM((PER_W,), jnp.float32),
        pltpu.SemaphoreType.DMA,
        pltpu.SemaphoreType.DMA,
    ],
)
def _lookup(ids_hbm, table_hbm, out_hbm, idx_v, val_v, out_v, sem0, sem1):
    wid = lax.axis_index("s") * NC + lax.axis_index("c")
    base = wid * PER_W
    pltpu.sync_copy(ids_hbm.at[pl.ds(base, PER_W)], idx_v)

    sems = (sem0, sem1)

    def fire(g, buf):
        goff = g * GROUP
        for j in range(CPG):
            pltpu.async_copy(
                table_hbm.at[idx_v.at[pl.ds(goff + j * CHUNK, CHUNK)]],
                val_v.at[buf].at[pl.ds(j * CHUNK, CHUNK)],
                sems[buf],
            )

    def drain(buf):
        pltpu.make_async_copy(
            table_hbm.at[pl.ds(0, GROUP)], val_v.at[buf], sems[buf]
        ).wait()

    def fixup(g, buf):
        goff = g * GROUP
        for v in range(GROUP // L):
            i16 = idx_v[pl.ds(goff + v * L, L)]
            v16 = val_v[buf, pl.ds(v * L, L)]
            r = jnp.where(
                i16 == NONEDGE_ID,
                jnp.float32(-jnp.inf),
                jnp.where(i16 == PADEDGE_ID, jnp.float32(0.0), v16),
            )
            out_v[pl.ds(goff + v * L, L)] = r

    fire(0, 0)
    fire(1, 1)

    def body(g, _):
        drain(0)
        fixup(g, 0)
        fire(g + 2, 0)
        drain(1)
        fixup(g + 1, 1)
        fire(g + 3, 1)
        return ()

    lax.fori_loop(0, (NG - 2) // 2, lambda i, c: body(i * 2, c), (), unroll=False)

    drain(0)
    fixup(NG - 2, 0)
    drain(1)
    fixup(NG - 1, 1)

    pltpu.sync_copy(out_v, out_hbm.at[pl.ds(base, PER_W)])


@jax.jit
def kernel(lattice_encoding, table):
    ids = lattice_encoding.reshape(N)
    tab = table.reshape(-1)
    out = _lookup(ids, tab)
    return out.reshape(ROWS, COLS)


# speedup vs baseline: 1.3749x; 1.0897x over previous
import functools

import jax
import jax.numpy as jnp
from jax import lax
from jax.experimental import pallas as pl
from jax.experimental.pallas import tpu as pltpu
from jax.experimental.pallas import tpu_sc as plsc

NONEDGE_ID = 0
PADEDGE_ID = 1

ROWS, COLS = 4096, 200
N = ROWS * COLS
VOCAB = 1000000
NC, NS, L = 2, 16, 16
NW = NC * NS
PER_W = N // NW
CHUNK = 128
CPG = 20
GROUP = CHUNK * CPG
NG = PER_W // GROUP

STAGE_TILES = 8
STAGE_SZ = VOCAB // STAGE_TILES
SCHUNK = 5000
NSC = STAGE_SZ // SCHUNK
NBUF = 2

assert PER_W % GROUP == 0
assert STAGE_SZ % 8 == 0 and SCHUNK % 8 == 0 and STAGE_SZ % SCHUNK == 0

_mesh = plsc.VectorSubcoreMesh(core_axis_name="c", subcore_axis_name="s")


@functools.partial(
    pl.kernel,
    out_type=jax.ShapeDtypeStruct((N,), jnp.float32),
    mesh=_mesh,
    scratch_types=[
        pltpu.VMEM((PER_W,), jnp.int32),
        pltpu.VMEM((PER_W,), jnp.float32),
        [pltpu.VMEM((SCHUNK,), jnp.float32) for _ in range(NBUF)],
        pltpu.VMEM((L,), jnp.float32),
        pltpu.VMEM_SHARED((VOCAB,), jnp.float32),
        pltpu.SemaphoreType.DMA,
        pltpu.SemaphoreType.DMA,
        pltpu.SemaphoreType.DMA,
    ],
)
def _lookup(ids_hbm, table_hbm, out_hbm, idx_v, out_v, bnc_v, patch_v,
            tab_sh, sem, sem_in, sem_out):
    cid = lax.axis_index("c")
    sid = lax.axis_index("s")
    wid = sid * NC + cid
    base = wid * PER_W

    pltpu.sync_copy(ids_hbm.at[pl.ds(base, PER_W)], idx_v)

    @pl.when(sid < STAGE_TILES)
    def _():
        toff = sid * STAGE_SZ

        def load(k):
            return pltpu.async_copy(
                table_hbm.at[pl.ds(toff + k * SCHUNK, SCHUNK)],
                bnc_v[k % NBUF], sem_in)

        loads = {k: load(k) for k in range(NBUF)}
        stores = {}
        for k in range(NSC):
            loads[k].wait()
            stores[k] = pltpu.async_copy(
                bnc_v[k % NBUF],
                tab_sh.at[pl.ds(toff + k * SCHUNK, SCHUNK)], sem_out)
            if k + NBUF < NSC:
                stores[k].wait()
                loads[k + NBUF] = load(k + NBUF)
        for k in range(NSC - NBUF, NSC):
            stores[k].wait()

    plsc.subcore_barrier()

    @pl.when(sid == 0)
    def _():
        pltpu.sync_copy(tab_sh.at[pl.ds(0, L)], patch_v)
        lane = lax.iota(jnp.int32, L)
        head = patch_v[...]
        head = jnp.where(lane == NONEDGE_ID, jnp.float32(-jnp.inf),
                         jnp.where(lane == PADEDGE_ID, jnp.float32(0.0), head))
        patch_v[...] = head
        pltpu.sync_copy(patch_v, tab_sh.at[pl.ds(0, L)])

    plsc.subcore_barrier()

    def fire_group(g, _):
        goff = g * GROUP
        for j in range(CPG):
            off = goff + j * CHUNK
            pltpu.async_copy(
                tab_sh.at[idx_v.at[pl.ds(off, CHUNK)]],
                out_v.at[pl.ds(off, CHUNK)],
                sem,
            )
        return ()

    lax.fori_loop(0, NG, fire_group, (), unroll=False)

    pltpu.make_async_copy(table_hbm.at[pl.ds(0, PER_W)], out_v, sem).wait()

    pltpu.sync_copy(out_v, out_hbm.at[pl.ds(base, PER_W)])


@jax.jit
def kernel(lattice_encoding, table):
    ids = lattice_encoding.reshape(N)
    tab = table.reshape(-1)
    out = _lookup(ids, tab)
    return out.reshape(ROWS, COLS)


# speedup vs baseline: 1.4583x; 1.0607x over previous
import functools

import jax
import jax.numpy as jnp
from jax import lax
from jax.experimental import pallas as pl
from jax.experimental.pallas import tpu as pltpu
from jax.experimental.pallas import tpu_sc as plsc

NONEDGE_ID = 0
PADEDGE_ID = 1

ROWS, COLS = 4096, 200
N = ROWS * COLS
VOCAB = 1000000
NC, NS, L = 2, 16, 16
NW = NC * NS
PER_W = N // NW
CHUNK = 128
CPG = 20
GROUP = CHUNK * CPG
NG = PER_W // GROUP

STRIPE = 62496
TAILO = STRIPE * NS
TAILN = VOCAB - TAILO
SCHUNK = 5208
NSC = STRIPE // SCHUNK
NBUF = 2

assert PER_W % GROUP == 0
assert STRIPE % 8 == 0 and SCHUNK % 8 == 0 and STRIPE % SCHUNK == 0
assert TAILN % 8 == 0 and TAILO % 8 == 0

_mesh = plsc.VectorSubcoreMesh(core_axis_name="c", subcore_axis_name="s")


@functools.partial(
    pl.kernel,
    out_type=jax.ShapeDtypeStruct((N,), jnp.float32),
    mesh=_mesh,
    scratch_types=[
        pltpu.VMEM((PER_W,), jnp.int32),
        pltpu.VMEM((PER_W,), jnp.float32),
        [pltpu.VMEM((SCHUNK,), jnp.float32) for _ in range(NBUF)],
        pltpu.VMEM((L,), jnp.float32),
        pltpu.VMEM_SHARED((VOCAB,), jnp.float32),
        pltpu.SemaphoreType.DMA,
        pltpu.SemaphoreType.DMA,
        pltpu.SemaphoreType.DMA,
    ],
)
def _lookup(ids_hbm, table_hbm, out_hbm, idx_v, out_v, bnc_v, patch_v,
            tab_sh, sem, sem_in, sem_out):
    cid = lax.axis_index("c")
    sid = lax.axis_index("s")
    wid = sid * NC + cid
    base = wid * PER_W

    pltpu.sync_copy(ids_hbm.at[pl.ds(base, PER_W)], idx_v)

    toff = sid * STRIPE

    def load(k):
        return pltpu.async_copy(
            table_hbm.at[pl.ds(toff + k * SCHUNK, SCHUNK)],
            bnc_v[k % NBUF], sem_in)

    loads = {k: load(k) for k in range(NBUF)}
    stores = {}
    for k in range(NSC):
        loads[k].wait()
        stores[k] = pltpu.async_copy(
            bnc_v[k % NBUF],
            tab_sh.at[pl.ds(toff + k * SCHUNK, SCHUNK)], sem_out)
        if k + NBUF < NSC:
            stores[k].wait()
            loads[k + NBUF] = load(k + NBUF)
    for k in range(NSC - NBUF, NSC):
        stores[k].wait()

    @pl.when(sid == 0)
    def _():
        pltpu.sync_copy(table_hbm.at[pl.ds(TAILO, TAILN)],
                        bnc_v[0].at[pl.ds(0, TAILN)])
        pltpu.sync_copy(bnc_v[0].at[pl.ds(0, TAILN)],
                        tab_sh.at[pl.ds(TAILO, TAILN)])
        pltpu.sync_copy(tab_sh.at[pl.ds(0, L)], patch_v)
        lane = lax.iota(jnp.int32, L)
        head = patch_v[...]
        head = jnp.where(lane == NONEDGE_ID, jnp.float32(-jnp.inf),
                         jnp.where(lane == PADEDGE_ID, jnp.float32(0.0), head))
        patch_v[...] = head
        pltpu.sync_copy(patch_v, tab_sh.at[pl.ds(0, L)])

    plsc.subcore_barrier()

    def fire_group(g, _):
        goff = g * GROUP
        for j in range(CPG):
            off = goff + j * CHUNK
            pltpu.async_copy(
                tab_sh.at[idx_v.at[pl.ds(off, CHUNK)]],
                out_v.at[pl.ds(off, CHUNK)],
                sem,
            )
        return ()

    lax.fori_loop(0, NG, fire_group, (), unroll=False)

    pltpu.make_async_copy(out_hbm.at[pl.ds(0, PER_W)], out_v, sem).wait()

    pltpu.sync_copy(out_v, out_hbm.at[pl.ds(base, PER_W)])


@jax.jit
def kernel(lattice_encoding, table):
    ids = lattice_encoding.reshape(N)
    tab = table.reshape(-1)
    out = _lookup(ids, tab)
    return out.reshape(ROWS, COLS)


# speedup vs baseline: 2.2794x; 1.5630x over previous
import functools

import jax
import jax.numpy as jnp
from jax import lax
from jax.experimental import pallas as pl
from jax.experimental.pallas import tpu as pltpu
from jax.experimental.pallas import tpu_sc as plsc

NONEDGE_ID = 0
PADEDGE_ID = 1

ROWS, COLS = 4096, 200
N = ROWS * COLS
VOCAB = 1000000
NC, NS, L = 2, 16, 16
NW = NC * NS
PER_W = N // NW
CHUNK = 128
CPG = 20
GROUP = CHUNK * CPG
NG = PER_W // GROUP

VOCAB_PAD = 1000448
STRIPE = VOCAB_PAD // NS
SCHUNK = 7816
NSC = STRIPE // SCHUNK
NBUF = 2

assert PER_W % GROUP == 0
assert STRIPE % 8 == 0 and SCHUNK % 8 == 0 and STRIPE % SCHUNK == 0

_mesh = plsc.VectorSubcoreMesh(core_axis_name="c", subcore_axis_name="s")


@functools.partial(
    pl.kernel,
    out_type=jax.ShapeDtypeStruct((N,), jnp.float32),
    mesh=_mesh,
    scratch_types=[
        pltpu.VMEM((PER_W,), jnp.int32),
        pltpu.VMEM((PER_W,), jnp.float32),
        [pltpu.VMEM((SCHUNK,), jnp.float32) for _ in range(NBUF)],
        pltpu.VMEM((L,), jnp.float32),
        pltpu.VMEM_SHARED((VOCAB_PAD,), jnp.float32),
        pltpu.SemaphoreType.DMA,
        pltpu.SemaphoreType.DMA,
        pltpu.SemaphoreType.DMA,
    ],
)
def _lookup(ids_hbm, table_hbm, out_hbm, idx_v, out_v, bnc_v, patch_v,
            tab_sh, sem, sem_in, sem_out):
    cid = lax.axis_index("c")
    sid = lax.axis_index("s")
    wid = sid * NC + cid
    base = wid * PER_W

    pltpu.sync_copy(ids_hbm.at[pl.ds(base, PER_W)], idx_v)

    toff = sid * STRIPE

    def load(k):
        return pltpu.async_copy(
            table_hbm.at[pl.ds(toff + k * SCHUNK, SCHUNK)],
            bnc_v[k % NBUF], sem_in)

    loads = {k: load(k) for k in range(NBUF)}
    stores = {}
    for k in range(NSC):
        loads[k].wait()
        stores[k] = pltpu.async_copy(
            bnc_v[k % NBUF],
            tab_sh.at[pl.ds(toff + k * SCHUNK, SCHUNK)], sem_out)
        if k + NBUF < NSC:
            stores[k].wait()
            loads[k + NBUF] = load(k + NBUF)
    for k in range(NSC - NBUF, NSC):
        stores[k].wait()

    @pl.when(sid == 0)
    def _():
        pltpu.sync_copy(tab_sh.at[pl.ds(0, L)], patch_v)
        lane = lax.iota(jnp.int32, L)
        head = patch_v[...]
        head = jnp.where(lane == NONEDGE_ID, jnp.float32(-jnp.inf),
                         jnp.where(lane == PADEDGE_ID, jnp.float32(0.0), head))
        patch_v[...] = head
        pltpu.sync_copy(patch_v, tab_sh.at[pl.ds(0, L)])

    plsc.subcore_barrier()

    def fire_group(g, _):
        goff = g * GROUP
        for j in range(CPG):
            off = goff + j * CHUNK
            pltpu.async_copy(
                tab_sh.at[idx_v.at[pl.ds(off, CHUNK)]],
                out_v.at[pl.ds(off, CHUNK)],
                sem,
            )
        return ()

    lax.fori_loop(0, NG, fire_group, (), unroll=False)

    pltpu.make_async_copy(out_hbm.at[pl.ds(0, PER_W)], out_v, sem).wait()

    pltpu.sync_copy(out_v, out_hbm.at[pl.ds(base, PER_W)])


@jax.jit
def kernel(lattice_encoding, table):
    ids = lattice_encoding.reshape(N)
    tab = jnp.pad(table, ((0, VOCAB_PAD - VOCAB), (0, 0))).reshape(-1)
    out = _lookup(ids, tab)
    return out.reshape(ROWS, COLS)
